# baseline (speedup 1.0000x reference)
import jax
import jax.numpy as jnp
from jax.experimental import pallas as pl


def _fused_body(a_ref, x_ref, wbd_ref, cbd_ref, bias_ref, cb_ref, o_ref):
    A = a_ref[...]
    n = A.shape[0]
    row = jax.lax.broadcasted_iota(jnp.int32, (n, n), 0)
    col = jax.lax.broadcasted_iota(jnp.int32, (n, n), 1)
    A0 = jnp.where(row == col, 0.0, A)
    deg = jnp.sum(A0, axis=1, keepdims=True)
    d = jnp.where(deg > 0, jax.lax.rsqrt(deg), 0.0)

    x = x_ref[...]
    t1 = jnp.dot(A0, x * d, preferred_element_type=jnp.float32)
    tx1 = -d * t1
    t2 = jnp.dot(A0, tx1 * d, preferred_element_type=jnp.float32)
    tx2 = -2.0 * d * t2 - x

    cat = jnp.concatenate([x, tx1, tx2], axis=1)
    out = jnp.dot(cat, wbd_ref[...], preferred_element_type=jnp.float32)
    out = jnp.maximum(out + bias_ref[...], 0.0)
    y = jnp.dot(out, cbd_ref[...], preferred_element_type=jnp.float32)
    o_ref[...] = jnp.maximum(y + cb_ref[0, 0], 0.0)


def kernel(X, A, W, b_gcn, conv_w, conv_b):
    B, N, _, T1 = X.shape
    K, _, T2 = W.shape
    Kc = conv_w.shape[2]
    pad = (Kc - 1) // 2

    x = X.reshape(B, N, T1).transpose(1, 0, 2).reshape(N, B * T1)
    eyeB = jnp.eye(B, dtype=X.dtype)
    wbd = jnp.concatenate([jnp.kron(eyeB, W[k]) for k in range(K)], axis=0)
    cw = conv_w.reshape(Kc)
    C = sum(cw[k] * jnp.eye(T2, k=pad - k, dtype=X.dtype) for k in range(Kc))
    cbd = jnp.kron(eyeB, C)
    bias = jnp.tile(b_gcn, B).reshape(1, B * T2)
    cb = conv_b.reshape(1, 1)

    y = pl.pallas_call(
        _fused_body,
        out_shape=jax.ShapeDtypeStruct((N, B * T2), X.dtype),
    )(A, x, wbd, cbd, bias, cb)
    return y.reshape(N, B, T2).transpose(1, 0, 2).reshape(B, N, 1, T2)
